# Initial kernel scaffold; baseline (speedup 1.0000x reference)
import jax
import jax.numpy as jnp
from jax.experimental import pallas as pl


def kernel(x, edge_index, W1, b1, W2, b2):
    raise NotImplementedError("write your pallas kernel here")


# speedup vs baseline: 16.2437x; 16.2437x over previous
import dataclasses
import functools

import jax
import jax.numpy as jnp
from jax import lax
from jax.experimental import pallas as pl
from jax.experimental.pallas import tpu as pltpu
from jax.experimental.pallas import tpu_sc as plsc

NC = 2
NS = 16
NW = NC * NS
CHUNK = 128


def _sc_mesh():
    return plsc.VectorSubcoreMesh(core_axis_name="c", subcore_axis_name="s")


def _sc_compiler_params():
    cp = pltpu.CompilerParams()
    if "needs_layout_passes" in pltpu.CompilerParams.__dataclass_fields__:
        cp = dataclasses.replace(cp, needs_layout_passes=False)
    return cp


@functools.lru_cache(maxsize=None)
def _make_deg(E, Np):
    n_chunks = E // CHUNK

    @functools.partial(
        pl.kernel,
        mesh=_sc_mesh(),
        compiler_params=_sc_compiler_params(),
        out_type=jax.ShapeDtypeStruct((NC, NS, Np), jnp.float32),
        scratch_types=[
            pltpu.VMEM((CHUNK,), jnp.int32),
            pltpu.VMEM((Np,), jnp.float32),
        ],
    )
    def deg_kernel(dst_hbm, out_hbm, dst_v, hist_v):
        c = lax.axis_index("c")
        s = lax.axis_index("s")
        wid = s * NC + c
        zeros = jnp.zeros((16,), jnp.float32)
        ones = jnp.ones((16,), jnp.float32)

        @pl.loop(0, Np, step=16)
        def _(i):
            hist_v[pl.ds(i, 16)] = zeros

        @pl.loop(wid, n_chunks, step=NW)
        def _(t):
            pltpu.sync_copy(dst_hbm.at[pl.ds(t * CHUNK, CHUNK)], dst_v)

            @pl.loop(0, CHUNK, step=16)
            def _(j):
                plsc.addupdate_scatter(hist_v, [dst_v[pl.ds(j, 16)]], ones)

        pltpu.sync_copy(hist_v, out_hbm.at[c].at[s])

    return deg_kernel


@functools.lru_cache(maxsize=None)
def _make_agg(E, Np, D):
    n_chunks = E // CHUNK
    rows = Np // NS

    @functools.partial(
        pl.kernel,
        mesh=_sc_mesh(),
        out_type=jax.ShapeDtypeStruct((NC, Np, D), jnp.float32),
        scratch_types=[
            pltpu.VMEM((CHUNK,), jnp.int32),
            pltpu.VMEM((CHUNK,), jnp.int32),
            pltpu.VMEM((CHUNK, D), jnp.float32),
            pltpu.VMEM_SHARED((Np, D), jnp.float32),
        ],
    )
    def agg_kernel(u_hbm, src_hbm, dst_hbm, zeros_hbm, out_hbm,
                   src_v, dst_v, rows_v, acc_sh):
        c = lax.axis_index("c")
        s = lax.axis_index("s")
        wid = s * NC + c
        pltpu.sync_copy(zeros_hbm.at[pl.ds(s * rows, rows)],
                        acc_sh.at[pl.ds(s * rows, rows)])
        plsc.subcore_barrier()

        @pl.loop(wid, n_chunks, step=NW)
        def _(t):
            pltpu.sync_copy(src_hbm.at[pl.ds(t * CHUNK, CHUNK)], src_v)
            pltpu.sync_copy(dst_hbm.at[pl.ds(t * CHUNK, CHUNK)], dst_v)
            pltpu.sync_copy(u_hbm.at[src_v], rows_v)
            pltpu.sync_copy(rows_v, acc_sh.at[dst_v], add=True)

        plsc.subcore_barrier()
        pltpu.sync_copy(acc_sh.at[pl.ds(s * rows, rows)],
                        out_hbm.at[c].at[pl.ds(s * rows, rows)])

    return agg_kernel


_BN = 1000


def _mm_body(x_ref, w_ref, o_ref):
    o_ref[...] = jnp.dot(x_ref[...], w_ref[...],
                         preferred_element_type=jnp.float32)


def _tc_matmul(x, W):
    Nn, K = x.shape
    D = W.shape[1]
    return pl.pallas_call(
        _mm_body,
        grid=(Nn // _BN,),
        in_specs=[pl.BlockSpec((_BN, K), lambda i: (i, 0)),
                  pl.BlockSpec((K, D), lambda i: (0, 0))],
        out_specs=pl.BlockSpec((_BN, D), lambda i: (i, 0)),
        out_shape=jax.ShapeDtypeStruct((Nn, D), jnp.float32),
    )(x, W)


def _scale_body(deg_ref, h_ref, o_ref):
    dinv = lax.rsqrt(deg_ref[...])
    o_ref[...] = h_ref[...] * dinv


def _tc_scale(deg, h):
    Nn, D = h.shape
    return pl.pallas_call(
        _scale_body,
        grid=(Nn // _BN,),
        in_specs=[pl.BlockSpec((_BN, 1), lambda i: (i, 0)),
                  pl.BlockSpec((_BN, D), lambda i: (i, 0))],
        out_specs=pl.BlockSpec((_BN, D), lambda i: (i, 0)),
        out_shape=jax.ShapeDtypeStruct((Nn, D), jnp.float32),
    )(deg, h)


def _layer_body(deg_ref, u_ref, p_ref, b_ref, w_ref, o_ref):
    dinv = lax.rsqrt(deg_ref[...])
    agg = u_ref[...] + p_ref[0] + p_ref[1]
    h = jnp.maximum(agg * dinv + b_ref[...], 0.0)
    o_ref[...] = jnp.dot(h, w_ref[...],
                         preferred_element_type=jnp.float32) * dinv


def _tc_layer(deg, u, p, b, W):
    Nn, D = u.shape
    D2 = W.shape[1]
    return pl.pallas_call(
        _layer_body,
        grid=(Nn // _BN,),
        in_specs=[pl.BlockSpec((_BN, 1), lambda i: (i, 0)),
                  pl.BlockSpec((_BN, D), lambda i: (i, 0)),
                  pl.BlockSpec((NC, _BN, D), lambda i: (0, i, 0)),
                  pl.BlockSpec((1, D), lambda i: (0, 0)),
                  pl.BlockSpec((D, D2), lambda i: (0, 0))],
        out_specs=pl.BlockSpec((_BN, D2), lambda i: (i, 0)),
        out_shape=jax.ShapeDtypeStruct((Nn, D2), jnp.float32),
    )(deg, u, p, b, W)


def _final_body(deg_ref, u_ref, p_ref, b_ref, o_ref):
    dinv = lax.rsqrt(deg_ref[...])
    agg = u_ref[...] + p_ref[0] + p_ref[1]
    h = jnp.maximum(agg * dinv + b_ref[...], 0.0)
    nrm = jnp.sqrt(jnp.sum(h * h, axis=1, keepdims=True))
    o_ref[...] = h / jnp.maximum(nrm, 1e-12)


def _tc_final(deg, u, p, b):
    Nn, D = u.shape
    return pl.pallas_call(
        _final_body,
        grid=(Nn // _BN,),
        in_specs=[pl.BlockSpec((_BN, 1), lambda i: (i, 0)),
                  pl.BlockSpec((_BN, D), lambda i: (i, 0)),
                  pl.BlockSpec((NC, _BN, D), lambda i: (0, i, 0)),
                  pl.BlockSpec((1, D), lambda i: (0, 0))],
        out_specs=pl.BlockSpec((_BN, D), lambda i: (i, 0)),
        out_shape=jax.ShapeDtypeStruct((Nn, D), jnp.float32),
    )(deg, u, p, b)


def kernel(x, edge_index, W1, b1, W2, b2):
    Nn, _ = x.shape
    Dh = W1.shape[1]
    E = edge_index.shape[1]
    src = edge_index[0]
    dst = edge_index[1]

    Np = -(-Nn // (NS * 8)) * (NS * 8)

    zerosD = jnp.zeros((Np, Dh), jnp.float32)

    degp = _make_deg(E, Np)(dst)
    h1 = _tc_matmul(x, W1)
    deg = (1.0 + degp.sum((0, 1))[:Nn])[:, None]

    u1 = _tc_scale(deg, h1)
    p1 = _make_agg(E, Np, Dh)(u1, src, dst, zerosD)
    u2 = _tc_layer(deg, u1, p1, b1.reshape(1, -1), W2)
    p2 = _make_agg(E, Np, Dh)(u2, src, dst, zerosD)
    return _tc_final(deg, u2, p2, b2.reshape(1, -1))
